# baseline (speedup 1.0000x reference)
import functools

import jax
import jax.numpy as jnp
from jax import lax
from jax.experimental import pallas as pl
from jax.experimental.pallas import tpu as pltpu
from jax.experimental.pallas import tpu_sc as plsc


@functools.lru_cache(maxsize=None)
def _build_gather(V, D, B, chunk):
    info = plsc.get_sparse_core_info()
    NC, NS = info.num_cores, info.num_subcores
    NW = NC * NS
    assert B % NW == 0
    b_per_w = B // NW
    assert b_per_w % chunk == 0 and chunk % 8 == 0
    nchunks = b_per_w // chunk
    assert nchunks % 2 == 0
    npairs = nchunks // 2
    mesh = plsc.VectorSubcoreMesh(core_axis_name="c", subcore_axis_name="s")

    @functools.partial(
        pl.kernel,
        mesh=mesh,
        out_type=jax.ShapeDtypeStruct((B, D), jnp.float32),
        scratch_types=[
            pltpu.VMEM((b_per_w,), jnp.int32),
            pltpu.VMEM((chunk, D), jnp.float32),
            pltpu.VMEM((chunk, D), jnp.float32),
            pltpu.SemaphoreType.DMA,
            pltpu.SemaphoreType.DMA,
            pltpu.SemaphoreType.DMA,
            pltpu.SemaphoreType.DMA,
        ],
    )
    def gather_kernel(
        table_hbm, idx_hbm, out_hbm,
        idx_v, rows_a, rows_b,
        gs_a, gs_b, os_a, os_b,
    ):
        wid = lax.axis_index("s") * NC + lax.axis_index("c")
        base = wid * b_per_w
        pltpu.sync_copy(idx_hbm.at[pl.ds(base, b_per_w)], idx_v)

        def g_copy(g, buf, sem):
            return pltpu.make_async_copy(
                table_hbm.at[idx_v.at[pl.ds(g * chunk, chunk)]], buf, sem
            )

        def w_copy(g, buf, sem):
            return pltpu.make_async_copy(
                buf, out_hbm.at[pl.ds(base + g * chunk, chunk)], sem
            )

        g_copy(0, rows_a, gs_a).start()

        def body(p, carry):
            g0 = 2 * p
            g1 = g0 + 1
            g_copy(g1, rows_b, gs_b).start()
            g_copy(g0, rows_a, gs_a).wait()
            w_copy(g0, rows_a, os_a).start()
            g_copy(g1, rows_b, gs_b).wait()
            w_copy(g1, rows_b, os_b).start()
            w_copy(g0, rows_a, os_a).wait()

            @pl.when(p + 1 < npairs)
            def _():
                g_copy(g0 + 2, rows_a, gs_a).start()

            w_copy(g1, rows_b, os_b).wait()
            return carry

        lax.fori_loop(0, npairs, body, 0, unroll=False)

    return gather_kernel


def kernel(x_gat_fin, edge_index):
    V, D = x_gat_fin.shape
    E = edge_index.shape[1]
    idx = edge_index.astype(jnp.int32).T.reshape(-1)
    out2 = _build_gather(V, D, 2 * E, 400)(x_gat_fin, idx)
    return out2.reshape(E, 2 * D)


# speedup vs baseline: 1.4510x; 1.4510x over previous
import functools

import jax
import jax.numpy as jnp
from jax import lax
from jax.experimental import pallas as pl
from jax.experimental.pallas import tpu as pltpu
from jax.experimental.pallas import tpu_sc as plsc


@functools.lru_cache(maxsize=None)
def _build_gather(V, D, E, chunk):
    info = plsc.get_sparse_core_info()
    NC, NS = info.num_cores, info.num_subcores
    NW = NC * NS
    assert E % NW == 0
    e_per_w = E // NW
    assert e_per_w % chunk == 0 and chunk % 8 == 0
    nchunks = e_per_w // chunk
    assert nchunks % 2 == 0
    npairs = nchunks // 2
    mesh = plsc.VectorSubcoreMesh(core_axis_name="c", subcore_axis_name="s")

    @functools.partial(
        pl.kernel,
        mesh=mesh,
        out_type=jax.ShapeDtypeStruct((E, 2, D), jnp.float32),
        scratch_types=[
            pltpu.VMEM((e_per_w,), jnp.int32),
            pltpu.VMEM((e_per_w,), jnp.int32),
            pltpu.VMEM((chunk, 2, D), jnp.float32),
            pltpu.VMEM((chunk, 2, D), jnp.float32),
            pltpu.SemaphoreType.DMA,
            pltpu.SemaphoreType.DMA,
            pltpu.SemaphoreType.DMA,
            pltpu.SemaphoreType.DMA,
            pltpu.SemaphoreType.DMA,
            pltpu.SemaphoreType.DMA,
        ],
    )
    def gather_kernel(
        table_hbm, idx_hbm, out_hbm,
        sidx_v, tidx_v, buf_a, buf_b,
        gss_a, gst_a, gss_b, gst_b, os_a, os_b,
    ):
        wid = lax.axis_index("s") * NC + lax.axis_index("c")
        base = wid * e_per_w
        pltpu.sync_copy(idx_hbm.at[pl.ds(base, e_per_w)], sidx_v)
        pltpu.sync_copy(idx_hbm.at[pl.ds(E + base, e_per_w)], tidx_v)

        def g_copies(g, buf, ssem, tsem):
            off = g * chunk
            return (
                pltpu.make_async_copy(
                    table_hbm.at[sidx_v.at[pl.ds(off, chunk)]],
                    buf.at[:, 0, :], ssem,
                ),
                pltpu.make_async_copy(
                    table_hbm.at[tidx_v.at[pl.ds(off, chunk)]],
                    buf.at[:, 1, :], tsem,
                ),
            )

        def w_copy(g, buf, sem):
            return pltpu.make_async_copy(
                buf, out_hbm.at[pl.ds(base + g * chunk, chunk)], sem
            )

        def start(copies):
            for c in copies:
                c.start()

        def wait(copies):
            for c in copies:
                c.wait()

        start(g_copies(0, buf_a, gss_a, gst_a))

        def body(p, carry):
            g0 = 2 * p
            g1 = g0 + 1
            start(g_copies(g1, buf_b, gss_b, gst_b))
            wait(g_copies(g0, buf_a, gss_a, gst_a))
            w_copy(g0, buf_a, os_a).start()
            wait(g_copies(g1, buf_b, gss_b, gst_b))
            w_copy(g1, buf_b, os_b).start()
            w_copy(g0, buf_a, os_a).wait()

            @pl.when(p + 1 < npairs)
            def _():
                start(g_copies(g0 + 2, buf_a, gss_a, gst_a))

            w_copy(g1, buf_b, os_b).wait()
            return carry

        lax.fori_loop(0, npairs, body, 0, unroll=False)

    return gather_kernel


def kernel(x_gat_fin, edge_index):
    V, D = x_gat_fin.shape
    E = edge_index.shape[1]
    idx = edge_index.astype(jnp.int32).reshape(-1)
    out = _build_gather(V, D, E, 200)(x_gat_fin, idx)
    return out.reshape(E, 2 * D)


# speedup vs baseline: 4.8993x; 3.3764x over previous
import functools

import jax
import jax.numpy as jnp
from jax import lax
from jax.experimental import pallas as pl
from jax.experimental.pallas import tpu as pltpu
from jax.experimental.pallas import tpu_sc as plsc


@functools.lru_cache(maxsize=None)
def _build_gather(V, D, E, chunk):
    info = plsc.get_sparse_core_info()
    NC, NS = info.num_cores, info.num_subcores
    NW = NC * NS
    assert E % NW == 0
    e_per_w = E // NW
    assert e_per_w % chunk == 0 and chunk % 8 == 0
    nchunks = e_per_w // chunk
    assert nchunks % 2 == 0
    npairs = nchunks // 2
    v_per_s = (V // NS) // 8 * 8
    v_last = V - v_per_s * (NS - 1)
    mesh = plsc.VectorSubcoreMesh(core_axis_name="c", subcore_axis_name="s")

    @functools.partial(
        pl.kernel,
        mesh=mesh,
        out_type=jax.ShapeDtypeStruct((E, 2 * D), jnp.float32),
        scratch_types=[
            pltpu.VMEM_SHARED((V, D), jnp.float32),
            pltpu.VMEM((e_per_w,), jnp.int32),
            pltpu.VMEM((e_per_w,), jnp.int32),
            pltpu.VMEM((chunk, D), jnp.float32),
            pltpu.VMEM((chunk, D), jnp.float32),
            pltpu.VMEM((chunk, D), jnp.float32),
            pltpu.VMEM((chunk, D), jnp.float32),
            pltpu.SemaphoreType.DMA,
            pltpu.SemaphoreType.DMA,
            pltpu.SemaphoreType.DMA,
            pltpu.SemaphoreType.DMA,
            pltpu.SemaphoreType.DMA,
            pltpu.SemaphoreType.DMA,
            pltpu.SemaphoreType.DMA,
            pltpu.SemaphoreType.DMA,
        ],
    )
    def gather_kernel(
        table_hbm, idx_hbm, out_hbm,
        table_s, sidx_v, tidx_v, srows_a, trows_a, srows_b, trows_b,
        gss_a, gst_a, gss_b, gst_b, oss_a, ost_a, oss_b, ost_b,
    ):
        sub = lax.axis_index("s")
        wid = sub * NC + lax.axis_index("c")
        base = wid * e_per_w

        row0 = sub * v_per_s
        nrow = jnp.where(sub == NS - 1, v_last, v_per_s)
        pltpu.sync_copy(
            table_hbm.at[pl.ds(row0, nrow)], table_s.at[pl.ds(row0, nrow)]
        )
        pltpu.sync_copy(idx_hbm.at[pl.ds(base, e_per_w)], sidx_v)
        pltpu.sync_copy(idx_hbm.at[pl.ds(E + base, e_per_w)], tidx_v)
        plsc.subcore_barrier()

        def g_copies(g, sbuf, tbuf, ssem, tsem):
            off = g * chunk
            return (
                pltpu.make_async_copy(
                    table_s.at[sidx_v.at[pl.ds(off, chunk)]], sbuf, ssem
                ),
                pltpu.make_async_copy(
                    table_s.at[tidx_v.at[pl.ds(off, chunk)]], tbuf, tsem
                ),
            )

        def w_copies(g, sbuf, tbuf, ssem, tsem):
            orow = base + g * chunk
            return (
                pltpu.make_async_copy(
                    sbuf, out_hbm.at[pl.ds(orow, chunk), pl.ds(0, D)], ssem
                ),
                pltpu.make_async_copy(
                    tbuf, out_hbm.at[pl.ds(orow, chunk), pl.ds(D, D)], tsem
                ),
            )

        def start(copies):
            for c in copies:
                c.start()

        def wait(copies):
            for c in copies:
                c.wait()

        start(g_copies(0, srows_a, trows_a, gss_a, gst_a))

        def body(p, carry):
            g0 = 2 * p
            g1 = g0 + 1
            start(g_copies(g1, srows_b, trows_b, gss_b, gst_b))
            wait(g_copies(g0, srows_a, trows_a, gss_a, gst_a))
            start(w_copies(g0, srows_a, trows_a, oss_a, ost_a))
            wait(g_copies(g1, srows_b, trows_b, gss_b, gst_b))
            start(w_copies(g1, srows_b, trows_b, oss_b, ost_b))
            wait(w_copies(g0, srows_a, trows_a, oss_a, ost_a))

            @pl.when(p + 1 < npairs)
            def _():
                start(g_copies(g0 + 2, srows_a, trows_a, gss_a, gst_a))

            wait(w_copies(g1, srows_b, trows_b, oss_b, ost_b))
            return carry

        lax.fori_loop(0, npairs, body, 0, unroll=False)

    return gather_kernel


def kernel(x_gat_fin, edge_index):
    V, D = x_gat_fin.shape
    E = edge_index.shape[1]
    idx = edge_index.astype(jnp.int32).reshape(-1)
    return _build_gather(V, D, E, 40)(x_gat_fin, idx)
